# baseline (speedup 1.0000x reference)
import functools

import jax
import jax.numpy as jnp
from jax import lax
from jax.experimental import pallas as pl
from jax.experimental.pallas import tpu as pltpu
from jax.experimental.pallas import tpu_sc as plsc

_GB = 128


def _make_gather(V, D, B, S):
    info = plsc.get_sparse_core_info()
    NC, NS = info.num_cores, info.num_subcores
    NW = NC * NS
    N = B * S
    DH, DL = D // 8, 8
    BH = B // _GB
    n_groups = S * BH
    assert n_groups % NW == 0
    g_per_w = n_groups // NW
    b_per_w = N // NW

    mesh = plsc.VectorSubcoreMesh(core_axis_name="c", subcore_axis_name="s")

    @functools.partial(
        pl.kernel,
        mesh=mesh,
        out_type=jax.ShapeDtypeStruct((S, DH, BH, DL, _GB), jnp.float32),
        compiler_params=pltpu.CompilerParams(use_tc_tiling_on_sc=False, needs_layout_passes=False, disable_bounds_checks=True),
        scratch_types=[
            pltpu.VMEM((b_per_w,), jnp.int32),
            pltpu.VMEM((_GB, D), jnp.float32),
            pltpu.VMEM((_GB, D), jnp.float32),
            pltpu.VMEM((D, _GB), jnp.float32),
            pltpu.VMEM((D, _GB), jnp.float32),
            pltpu.SemaphoreType.DMA,
            pltpu.SemaphoreType.DMA,
        ],
    )
    def gather_kernel(
        table_hbm, idx_hbm, out_hbm, idx_v, g_a, g_b, t_a, t_b, sem_g, sem_o
    ):
        g_bufs = (g_a, g_b)
        t_bufs = (t_a, t_b)
        wid = lax.axis_index("s") * NC + lax.axis_index("c")
        base = wid * b_per_w
        g0 = wid * g_per_w

        pltpu.sync_copy(idx_hbm.at[pl.ds(base, b_per_w)], idx_v)

        lane = lax.iota(jnp.int32, 16)

        def gather_copy(i, b):
            return pltpu.make_async_copy(
                table_hbm.at[idx_v.at[pl.ds(i * _GB, _GB)]], g_bufs[b], sem_g
            )

        def out_copies(i, b):
            g = g0 + i
            s = g // BH
            b_hi = g % BH
            return [
                pltpu.make_async_copy(
                    t_bufs[b].at[pl.ds(dh * DL, DL)],
                    out_hbm.at[s, dh, b_hi],
                    sem_o,
                )
                for dh in range(DH)
            ]

        rots = [(lane + r) & 15 for r in range(16)]
        zero16 = lane & 0

        def transpose(b):
            g_ref = g_bufs[b]
            t_ref = t_bufs[b]

            def blk(i0_blk, carry):
                i0 = i0_blk * 16
                rows = [i0 + rot for rot in rots]
                for d0 in range(0, D, 32):
                    cols = [d0 + lane, d0 + 16 + lane]
                    vs = [
                        plsc.load_gather(g_ref, [rows[r], cols[c]])
                        for c in range(2)
                        for r in range(16)
                    ]
                    for c in range(2):
                        for r in range(16):
                            plsc.store_scatter(
                                t_ref, [cols[c], rows[r]], vs[c * 16 + r]
                            )
                return carry

            lax.fori_loop(0, _GB // 16, blk, 0)

        gather_copy(0, 0).start()

        def body(gg, carry):
            for b in range(2):
                i = gg * 2 + b
                b2 = 1 - b
                gather_copy(i, b).wait()

                @pl.when(i + 1 < g_per_w)
                def _():
                    gather_copy(i + 1, b2).start()

                @pl.when(i >= 2)
                def _():
                    for c in out_copies(i - 2, b):
                        c.wait()

                transpose(b)
                for c in out_copies(i, b):
                    c.start()
            return carry

        lax.fori_loop(0, g_per_w // 2, body, 0)

        for j in (g_per_w - 2, g_per_w - 1):
            for c in out_copies(j, j % 2):
                c.wait()

    return gather_kernel


def kernel(inputs, table):
    B, S = inputs.shape
    V, D = table.shape
    idx = inputs.T.reshape(-1).astype(jnp.int32)
    gather = _make_gather(V, D, B, S)
    z = gather(table, idx)
    return z.transpose(2, 4, 0, 1, 3).reshape(B, S, D)


# speedup vs baseline: 1.0260x; 1.0260x over previous
import functools

import jax
import jax.numpy as jnp
from jax import lax
from jax.experimental import pallas as pl
from jax.experimental.pallas import tpu as pltpu
from jax.experimental.pallas import tpu_sc as plsc

_GB = 128


def _make_gather(V, D, B, S):
    info = plsc.get_sparse_core_info()
    NC, NS = info.num_cores, info.num_subcores
    NW = NC * NS
    N = B * S
    DH, DL = D // 8, 8
    BH = B // _GB
    n_groups = S * BH
    assert n_groups % NW == 0
    g_per_w = n_groups // NW
    b_per_w = N // NW

    mesh = plsc.VectorSubcoreMesh(core_axis_name="c", subcore_axis_name="s")

    @functools.partial(
        pl.kernel,
        mesh=mesh,
        out_type=jax.ShapeDtypeStruct((S, DH, BH, DL, _GB), jnp.float32),
        compiler_params=pltpu.CompilerParams(use_tc_tiling_on_sc=False, needs_layout_passes=False, disable_bounds_checks=True),
        scratch_types=[
            pltpu.VMEM((b_per_w,), jnp.int32),
            pltpu.VMEM((_GB, D), jnp.float32),
            pltpu.VMEM((_GB, D), jnp.float32),
            pltpu.VMEM((D, _GB), jnp.float32),
            pltpu.VMEM((D, _GB), jnp.float32),
            pltpu.SemaphoreType.DMA,
            pltpu.SemaphoreType.DMA,
        ],
    )
    def gather_kernel(
        table_hbm, idx_hbm, out_hbm, idx_v, g_a, g_b, t_a, t_b, sem_g, sem_o
    ):
        g_bufs = (g_a, g_b)
        t_bufs = (t_a, t_b)
        wid = lax.axis_index("s") * NC + lax.axis_index("c")
        base = wid * b_per_w
        g0 = wid * g_per_w

        pltpu.sync_copy(idx_hbm.at[pl.ds(base, b_per_w)], idx_v)

        lane = lax.iota(jnp.int32, 16)

        def gather_copy(i, b):
            return pltpu.make_async_copy(
                table_hbm.at[idx_v.at[pl.ds(i * _GB, _GB)]], g_bufs[b], sem_g
            )

        def out_copies(i, b):
            g = g0 + i
            s = g // BH
            b_hi = g % BH
            return [
                pltpu.make_async_copy(
                    t_bufs[b].at[pl.ds(dh * DL, DL)],
                    out_hbm.at[s, dh, b_hi],
                    sem_o,
                )
                for dh in range(DH)
            ]

        rots = [(lane + r) & 15 for r in range(16)]
        zero16 = lane & 0

        def transpose(b):
            g_ref = g_bufs[b]
            t_ref = t_bufs[b]

            def blk(i0_blk, carry):
                i0 = i0_blk * 16
                rows = [i0 + rot for rot in rots]
                for d0 in range(0, D, 16):
                    col = d0 + lane
                    vs = [plsc.load_gather(g_ref, [rows[r], col]) for r in range(16)]
                    for r in range(16):
                        plsc.store_scatter(t_ref, [col, rows[r]], vs[r])
                return carry

            lax.fori_loop(0, _GB // 16, blk, 0)

        gather_copy(0, 0).start()

        def body(gg, carry):
            for b in range(2):
                i = gg * 2 + b
                b2 = 1 - b
                gather_copy(i, b).wait()

                @pl.when(i + 1 < g_per_w)
                def _():
                    gather_copy(i + 1, b2).start()

                @pl.when(i >= 2)
                def _():
                    for c in out_copies(i - 2, b):
                        c.wait()

                transpose(b)
                for c in out_copies(i, b):
                    c.start()
            return carry

        lax.fori_loop(0, g_per_w // 2, body, 0)

        for j in (g_per_w - 2, g_per_w - 1):
            for c in out_copies(j, j % 2):
                c.wait()

    return gather_kernel


def kernel(inputs, table):
    B, S = inputs.shape
    V, D = table.shape
    idx = inputs.T.reshape(-1).astype(jnp.int32)
    gather = _make_gather(V, D, B, S)
    z = gather(table, idx)
    return z.transpose(2, 4, 0, 1, 3).reshape(B, S, D)
